# baseline (speedup 1.0000x reference)
import functools

import jax
import jax.numpy as jnp
from jax import lax
from jax.experimental import pallas as pl
from jax.experimental.pallas import tpu as pltpu
from jax.experimental.pallas import tpu_sc as plsc

NC = 2
NS = 16
LANES = 16
NSLOT = 4
LOOK = 3
ROWW = 80


def _tc_h_and_coeffs(x, W, Ms16, Md16):
    n, cin = x.shape
    ho = W.shape[1]
    half = ho // 2
    r = 1000 if n % 1000 == 0 else n

    def body(x_ref, w_ref, ms_ref, md_ref, hs_ref, ad_ref):
        h = jnp.dot(x_ref[...], w_ref[...], preferred_element_type=jnp.float32)
        asf = jnp.dot(h, ms_ref[...], preferred_element_type=jnp.float32)
        adf = jnp.dot(h, md_ref[...], preferred_element_type=jnp.float32)
        hs_ref[0] = jnp.concatenate([asf[:, :16], h[:, :half]], axis=1)
        hs_ref[1] = jnp.concatenate([asf[:, 16:], h[:, half:]], axis=1)
        ad_ref[0] = adf[:, :16]
        ad_ref[1] = adf[:, 16:]

    return pl.pallas_call(
        body,
        grid=(n // r,),
        in_specs=[
            pl.BlockSpec((r, cin), lambda i: (i, 0)),
            pl.BlockSpec((cin, ho), lambda i: (0, 0)),
            pl.BlockSpec((cin, 32), lambda i: (0, 0)),
            pl.BlockSpec((cin, 32), lambda i: (0, 0)),
        ],
        out_specs=[
            pl.BlockSpec((2, r, ROWW), lambda i: (0, i, 0)),
            pl.BlockSpec((2, r, 16), lambda i: (0, i, 0)),
        ],
        out_shape=[
            jax.ShapeDtypeStruct((2, n, ROWW), jnp.float32),
            jax.ShapeDtypeStruct((2, n, 16), jnp.float32),
        ],
    )(x, W, Ms16, Md16)


def _make_sc_edge_kernel(n, e, heads, out_dim):
    half = heads * out_dim // 2
    hpc = heads // 2
    ept = e // NS
    c = 8
    for cand in range(160, 7, -8):
        if ept % cand == 0:
            c = cand
            break
    rd = 80
    assert n % rd == 0
    n_rchunks = n // rd
    rchunks_per_tile = -(-n_rchunks // NS)

    mesh = plsc.VectorSubcoreMesh(
        core_axis_name="c", subcore_axis_name="s", num_cores=NC, num_subcores=NS)

    @functools.partial(
        pl.kernel,
        out_type=jax.ShapeDtypeStruct((NC * n, half), jnp.float32),
        mesh=mesh,
        compiler_params=pltpu.CompilerParams(use_tc_tiling_on_sc=False),
        scratch_types=[
            pltpu.VMEM_SHARED((n, ROWW), jnp.float32),
            pltpu.VMEM((NSLOT, c), jnp.int32),
            pltpu.VMEM((NSLOT, c), jnp.int32),
            pltpu.VMEM((NSLOT, c), jnp.int32),
            pltpu.VMEM((NSLOT, c, 16), jnp.float32),
            pltpu.VMEM((NSLOT, c, ROWW), jnp.float32),
            pltpu.VMEM((rd, ROWW), jnp.float32),
            pltpu.VMEM((rd, half), jnp.float32),
            pltpu.VMEM((half,), jnp.float32),
            pltpu.SemaphoreType.DMA((NSLOT,)),
            pltpu.SemaphoreType.DMA((NSLOT,)),
        ],
    )
    def edge_kernel(hs, adstp, srcv, dstv, biash, z80,
                    out, acc, dstbuf, hsrcbuf, hdstbuf, bg,
                    hsbuf, divbuf, outbuf, biasv, gsem, ssem):
        cid = lax.axis_index("c")
        sid = lax.axis_index("s")

        def zero_chunk(i, carry):
            idx = sid + NS * i

            @pl.when(idx < n_rchunks)
            def _():
                pltpu.sync_copy(z80, acc.at[pl.ds(idx * rd, rd)])

            return carry

        lax.fori_loop(0, rchunks_per_tile, zero_chunk, 0)
        pltpu.sync_copy(biash.at[pl.ds(cid * half, half)], biasv)

        ebase = sid * ept
        rowoff = cid * n
        nchunks = ept // c

        def fire_gathers(q, chunk):
            base = ebase + chunk * c
            pltpu.sync_copy(srcv.at[pl.ds(base, c)], hsrcbuf.at[q])
            pltpu.sync_copy(dstv.at[pl.ds(base, c)], dstbuf.at[q])
            for j in range(c // LANES):
                sl16 = pl.ds(j * LANES, LANES)
                hsrcbuf[q, sl16] = hsrcbuf[q, sl16] + rowoff
                hdstbuf[q, sl16] = dstbuf[q, sl16] + rowoff
            pltpu.async_copy(adstp.at[hdstbuf.at[q]], bg.at[q], gsem.at[q])
            pltpu.async_copy(hs.at[hsrcbuf.at[q]], hsbuf.at[q], gsem.at[q])

        def wait_gathers(q):
            pltpu.make_async_copy(adstp.at[hdstbuf.at[q]], bg.at[q], gsem.at[q]).wait()
            pltpu.make_async_copy(hs.at[hsrcbuf.at[q]], hsbuf.at[q], gsem.at[q]).wait()

        def fire_scatters(q):
            pltpu.async_copy(hsbuf.at[q], acc.at[dstbuf.at[q]], ssem.at[q], add=True)

        def wait_scatters(q):
            pltpu.make_async_copy(hsbuf.at[q], acc.at[dstbuf.at[q]], ssem.at[q]).wait()

        def compute(q):
            def edge_body(r, carry2):
                ev = hsbuf[q, r, pl.ds(0, 16)] + bg[q, r]
                ev = jnp.where(ev > 0.0, ev, 0.2 * ev)
                p = jnp.exp(ev)
                hsbuf[q, r, pl.ds(0, 16)] = p
                for k in range(hpc):
                    sl = pl.ds(16 + k * out_dim, out_dim)
                    hsbuf[q, r, sl] = hsbuf[q, r, sl] * p[k]
                return carry2

            lax.fori_loop(0, c, edge_body, 0, unroll=16)

        plsc.subcore_barrier()
        for pq in range(LOOK):
            fire_gathers(pq, pq)

        def chunk_body(i, carry):
            q = lax.rem(i, NSLOT)
            qf = lax.rem(i + LOOK, NSLOT)

            @pl.when(i + LOOK < nchunks)
            def _():
                @pl.when(i + LOOK >= NSLOT)
                def _():
                    wait_scatters(qf)

                fire_gathers(qf, i + LOOK)

            wait_gathers(q)
            compute(q)
            fire_scatters(q)
            return carry

        lax.fori_loop(0, nchunks, chunk_body, 0)
        for qq in range(NSLOT):
            wait_scatters((nchunks - 1 - qq) % NSLOT)
        plsc.subcore_barrier()

        def div_chunk(i, carry):
            idx = sid + NS * i

            @pl.when(idx < n_rchunks)
            def _():
                rr = idx * rd
                pltpu.sync_copy(acc.at[pl.ds(rr, rd)], divbuf)

                def row_body(r, carry2):
                    svr = divbuf[r, pl.ds(0, 16)]
                    for k in range(hpc):
                        sv = svr[k] + 1e-16
                        slo = pl.ds(k * out_dim, out_dim)
                        sli = pl.ds(16 + k * out_dim, out_dim)
                        outbuf[r, slo] = divbuf[r, sli] / sv + biasv[slo]
                    return carry2

                lax.fori_loop(0, rd, row_body, 0)
                pltpu.sync_copy(outbuf, out.at[pl.ds(rowoff + rr, rd)])

            return carry

        lax.fori_loop(0, rchunks_per_tile, div_chunk, 0)

    return edge_kernel


def kernel(x, edge_index, W, att_src, att_dst, bias):
    n, cin = x.shape
    heads, out_dim = att_src.shape
    ho = heads * out_dim
    e = edge_index.shape[1]

    hpc = heads // NC
    j = jnp.arange(ho)
    hd = j // out_dim
    col = (hd // hpc) * 16 + hd % hpc
    ms32 = jnp.zeros((ho, 32), jnp.float32).at[j, col].set(att_src.reshape(-1))
    md32 = jnp.zeros((ho, 32), jnp.float32).at[j, col].set(att_dst.reshape(-1))

    hsp, adp = _tc_h_and_coeffs(x, W, ms32, md32)
    hsflat = hsp.reshape(NC * n, ROWW)
    adstp = adp.reshape(NC * n, 16)

    src = edge_index[0]
    dst = edge_index[1]
    z80 = jnp.zeros((80, ROWW), jnp.float32)

    edge_kernel = _make_sc_edge_kernel(n, e, heads, out_dim)
    oc = edge_kernel(hsflat, adstp, src, dst, bias, z80)
    return jnp.concatenate([oc[:n], oc[n:]], axis=1)


# speedup vs baseline: 1.3066x; 1.3066x over previous
import functools

import jax
import jax.numpy as jnp
from jax import lax
from jax.experimental import pallas as pl
from jax.experimental.pallas import tpu as pltpu
from jax.experimental.pallas import tpu_sc as plsc

NC = 2
NS = 16
LANES = 16
NSLOT = 4
LOOK = 3
ROWW = 80


def _tc_h_and_coeffs(x, W, Ms16, Md16):
    n, cin = x.shape
    ho = W.shape[1]
    half = ho // 2
    r = 1000 if n % 1000 == 0 else n

    def body(x_ref, w_ref, ms_ref, md_ref, hs_ref, ad_ref):
        h = jnp.dot(x_ref[...], w_ref[...], preferred_element_type=jnp.float32)
        asf = jnp.dot(h, ms_ref[...], preferred_element_type=jnp.float32)
        adf = jnp.dot(h, md_ref[...], preferred_element_type=jnp.float32)
        hs_ref[0] = jnp.concatenate([asf[:, :16], h[:, :half]], axis=1)
        hs_ref[1] = jnp.concatenate([asf[:, 16:], h[:, half:]], axis=1)
        ad_ref[0] = adf[:, :16]
        ad_ref[1] = adf[:, 16:]

    return pl.pallas_call(
        body,
        grid=(n // r,),
        in_specs=[
            pl.BlockSpec((r, cin), lambda i: (i, 0)),
            pl.BlockSpec((cin, ho), lambda i: (0, 0)),
            pl.BlockSpec((cin, 32), lambda i: (0, 0)),
            pl.BlockSpec((cin, 32), lambda i: (0, 0)),
        ],
        out_specs=[
            pl.BlockSpec((2, r, ROWW), lambda i: (0, i, 0)),
            pl.BlockSpec((2, r, 16), lambda i: (0, i, 0)),
        ],
        out_shape=[
            jax.ShapeDtypeStruct((2, n, ROWW), jnp.float32),
            jax.ShapeDtypeStruct((2, n, 16), jnp.float32),
        ],
    )(x, W, Ms16, Md16)


def _make_sc_edge_kernel(n, e, heads, out_dim):
    half = heads * out_dim // 2
    hpc = heads // 2
    ept = e // NS
    c = 8
    for cand in range(160, 7, -8):
        if ept % cand == 0:
            c = cand
            break
    rd = 80
    assert n % rd == 0
    n_rchunks = n // rd
    rchunks_per_tile = -(-n_rchunks // NS)

    mesh = plsc.VectorSubcoreMesh(
        core_axis_name="c", subcore_axis_name="s", num_cores=NC, num_subcores=NS)

    @functools.partial(
        pl.kernel,
        out_type=jax.ShapeDtypeStruct((NC * n, half), jnp.float32),
        mesh=mesh,
        compiler_params=pltpu.CompilerParams(use_tc_tiling_on_sc=False),
        scratch_types=[
            pltpu.VMEM_SHARED((n, ROWW), jnp.float32),
            pltpu.VMEM((NSLOT, c), jnp.int32),
            pltpu.VMEM((NSLOT, c), jnp.int32),
            pltpu.VMEM((NSLOT, c), jnp.int32),
            pltpu.VMEM((NSLOT, c, 16), jnp.float32),
            pltpu.VMEM((NSLOT, c, ROWW), jnp.float32),
            pltpu.VMEM((rd, ROWW), jnp.float32),
            pltpu.VMEM((rd, half), jnp.float32),
            pltpu.VMEM((half,), jnp.float32),
            pltpu.SemaphoreType.DMA((NSLOT,)),
            pltpu.SemaphoreType.DMA((NSLOT,)),
        ],
    )
    def edge_kernel(hs, adstp, srcv, dstv, biash, z80,
                    out, acc, dstbuf, hsrcbuf, hdstbuf, bg,
                    hsbuf, divbuf, outbuf, biasv, gsem, ssem):
        cid = lax.axis_index("c")
        sid = lax.axis_index("s")

        def zero_chunk(i, carry):
            idx = sid + NS * i

            @pl.when(idx < n_rchunks)
            def _():
                pltpu.sync_copy(z80, acc.at[pl.ds(idx * rd, rd)])

            return carry

        lax.fori_loop(0, rchunks_per_tile, zero_chunk, 0)
        pltpu.sync_copy(biash.at[pl.ds(cid * half, half)], biasv)

        ebase = sid * ept
        rowoff = cid * n
        nchunks = ept // c

        def fire_gathers(q, chunk):
            base = ebase + chunk * c
            pltpu.sync_copy(srcv.at[pl.ds(base, c)], hsrcbuf.at[q])
            pltpu.sync_copy(dstv.at[pl.ds(base, c)], dstbuf.at[q])
            for j in range(c // LANES):
                sl16 = pl.ds(j * LANES, LANES)
                hsrcbuf[q, sl16] = hsrcbuf[q, sl16] + rowoff
                hdstbuf[q, sl16] = dstbuf[q, sl16] + rowoff
            pltpu.async_copy(adstp.at[hdstbuf.at[q]], bg.at[q], gsem.at[q])
            pltpu.async_copy(hs.at[hsrcbuf.at[q]], hsbuf.at[q], gsem.at[q])

        def wait_gathers(q):
            pltpu.make_async_copy(adstp.at[hdstbuf.at[q]], bg.at[q], gsem.at[q]).wait()
            pltpu.make_async_copy(hs.at[hsrcbuf.at[q]], hsbuf.at[q], gsem.at[q]).wait()

        def fire_scatters(q):
            pltpu.async_copy(hsbuf.at[q], acc.at[dstbuf.at[q]], ssem.at[q], add=True)

        def wait_scatters(q):
            pltpu.make_async_copy(hsbuf.at[q], acc.at[dstbuf.at[q]], ssem.at[q]).wait()

        def compute(q):
            def edge_body(r, carry2):
                ev = hsbuf[q, r, pl.ds(0, 16)] + bg[q, r]
                ev = jnp.where(ev > 0.0, ev, 0.2 * ev)
                p = jnp.exp(ev)
                hsbuf[q, r, pl.ds(0, 16)] = p
                for k in range(hpc):
                    sl = pl.ds(16 + k * out_dim, out_dim)
                    hsbuf[q, r, sl] = hsbuf[q, r, sl] * p[k]
                return carry2

            lax.fori_loop(0, c, edge_body, 0, unroll=8)

        plsc.subcore_barrier()
        for pq in range(LOOK):
            fire_gathers(pq, pq)

        def chunk_body(i, carry):
            q = lax.rem(i, NSLOT)
            qf = lax.rem(i + LOOK, NSLOT)

            @pl.when(i + LOOK < nchunks)
            def _():
                @pl.when(i + LOOK >= NSLOT)
                def _():
                    wait_scatters(qf)

                fire_gathers(qf, i + LOOK)

            wait_gathers(q)
            compute(q)
            fire_scatters(q)
            return carry

        lax.fori_loop(0, nchunks, chunk_body, 0)
        for qq in range(NSLOT):
            wait_scatters((nchunks - 1 - qq) % NSLOT)
        plsc.subcore_barrier()

        def div_chunk(i, carry):
            idx = sid + NS * i

            @pl.when(idx < n_rchunks)
            def _():
                rr = idx * rd
                pltpu.sync_copy(acc.at[pl.ds(rr, rd)], divbuf)

                def row_body(r, carry2):
                    svr = divbuf[r, pl.ds(0, 16)]
                    for k in range(hpc):
                        sv = svr[k] + 1e-16
                        slo = pl.ds(k * out_dim, out_dim)
                        sli = pl.ds(16 + k * out_dim, out_dim)
                        outbuf[r, slo] = divbuf[r, sli] / sv + biasv[slo]
                    return carry2

                lax.fori_loop(0, rd, row_body, 0)
                pltpu.sync_copy(outbuf, out.at[pl.ds(rowoff + rr, rd)])

            return carry

        lax.fori_loop(0, rchunks_per_tile, div_chunk, 0)

    return edge_kernel


def kernel(x, edge_index, W, att_src, att_dst, bias):
    n, cin = x.shape
    heads, out_dim = att_src.shape
    ho = heads * out_dim
    e = edge_index.shape[1]

    hpc = heads // NC
    j = jnp.arange(ho)
    hd = j // out_dim
    col = (hd // hpc) * 16 + hd % hpc
    ms32 = jnp.zeros((ho, 32), jnp.float32).at[j, col].set(att_src.reshape(-1))
    md32 = jnp.zeros((ho, 32), jnp.float32).at[j, col].set(att_dst.reshape(-1))

    hsp, adp = _tc_h_and_coeffs(x, W, ms32, md32)
    hsflat = hsp.reshape(NC * n, ROWW)
    adstp = adp.reshape(NC * n, 16)

    src = edge_index[0]
    dst = edge_index[1]
    z80 = jnp.zeros((80, ROWW), jnp.float32)

    edge_kernel = _make_sc_edge_kernel(n, e, heads, out_dim)
    oc = edge_kernel(hsflat, adstp, src, dst, bias, z80)
    return jnp.concatenate([oc[:n], oc[n:]], axis=1)
